# baseline (speedup 1.0000x reference)
import functools

import jax
import jax.numpy as jnp
from jax import lax
from jax.experimental import pallas as pl
from jax.experimental.pallas import tpu as pltpu
from jax.experimental.pallas import tpu_sc as plsc

M_NUM = 6000
D_NUM = 4000
VOCAB = M_NUM + D_NUM
FEA = 128
N_LISTS = 2
BATCH = 4096
NEI = 32

_BM = 400
_NBM_M = M_NUM // _BM
_NBM_D = D_NUM // _BM

_NC = 2
_NS = 16
_NW = _NC * _NS
_B_TOT = N_LISTS * BATCH * NEI
_BPW = _B_TOT // _NW
_CH = 128


def _proj_body(x_ref, w_ref, b_ref, o_ref):
    x = x_ref[...].astype(jnp.bfloat16)
    o_ref[...] = jnp.maximum(
        jnp.dot(x, w_ref[...], preferred_element_type=jnp.float32)
        + b_ref[...],
        0.0,
    )


def _project(x, w, b, bm):
    m, k = x.shape
    return pl.pallas_call(
        _proj_body,
        grid=(m // bm,),
        in_specs=[
            pl.BlockSpec((bm, k), lambda i: (i, 0)),
            pl.BlockSpec((k, FEA), lambda i: (0, 0)),
            pl.BlockSpec((1, FEA), lambda i: (0, 0)),
        ],
        out_specs=pl.BlockSpec((bm, FEA), lambda i: (i, 0)),
        out_shape=jax.ShapeDtypeStruct((m, FEA), jnp.float32),
    )(x, w.astype(jnp.bfloat16), b.reshape(1, FEA))


def _project_all(m_sim, d_sim, W, b):
    t_m = _project(m_sim, W[:M_NUM], b, 1000)
    t_d = _project(d_sim, W[M_NUM:], b, 1000)
    return jnp.concatenate([t_m, t_d], axis=0)


_CPW = 32
_ROWS3 = _B_TOT // _CH


def _gather_body(table_hbm, idx_hbm, out_hbm, idx_v, rows_v, gsems, ssems):
    wid = lax.axis_index("s") * _NC + lax.axis_index("c")
    base3 = wid * (_BPW // _CH)
    pltpu.sync_copy(idx_hbm.at[wid], idx_v)

    def _gather_start(j, p):
        pltpu.async_copy(table_hbm.at[idx_v.at[j, 0]], rows_v.at[p, 0], gsems.at[p])
        pltpu.async_copy(table_hbm.at[idx_v.at[j, 1]], rows_v.at[p, 1], gsems.at[p])

    def _gather_wait(j, p):
        pltpu.make_async_copy(
            table_hbm.at[idx_v.at[j, 0]], rows_v.at[p, 0], gsems.at[p]
        ).wait()
        pltpu.make_async_copy(
            table_hbm.at[idx_v.at[j, 1]], rows_v.at[p, 1], gsems.at[p]
        ).wait()

    def _store_start(j, p):
        pltpu.async_copy(
            rows_v.at[p], out_hbm.at[pl.ds(base3 + 2 * j, 2)], ssems.at[p]
        )

    def _store_wait(j, p):
        pltpu.make_async_copy(
            rows_v.at[p], out_hbm.at[pl.ds(base3 + 2 * j, 2)], ssems.at[p]
        ).wait()

    _gather_start(0, 0)

    def body(t, carry):
        for p in range(3):
            j = 3 * t + p
            pn = (p + 1) % 3

            @pl.when(j >= 2)
            def _():
                _store_wait(j - 2, pn)

            _gather_start(j + 1, pn)
            _gather_wait(j, p)
            _store_start(j, p)
        return carry

    lax.fori_loop(0, (_CPW - 2) // 3, body, 0)
    _store_wait(28, 1)
    _gather_start(31, 1)
    _gather_wait(30, 0)
    _store_start(30, 0)
    _store_wait(29, 2)
    _gather_wait(31, 1)
    _store_start(31, 1)
    _store_wait(30, 0)
    _store_wait(31, 1)


_gather = functools.partial(
    pl.kernel,
    out_type=jax.ShapeDtypeStruct((_ROWS3, _CH, FEA), jnp.float32),
    mesh=plsc.VectorSubcoreMesh(core_axis_name="c", subcore_axis_name="s"),
    scratch_types=[
        pltpu.VMEM((_CPW, 2, _CH), jnp.int32),
        pltpu.VMEM((3, 2, _CH, FEA), jnp.float32),
        pltpu.SemaphoreType.DMA((3,)),
        pltpu.SemaphoreType.DMA((3,)),
    ],
)(_gather_body)


def kernel(m_sim, d_sim, nei_node_list, W, b):
    table = _project_all(m_sim, d_sim, W, b)
    idx = nei_node_list.reshape(_NW, _CPW, 2, _CH)
    out = _gather(table, idx)
    return out.reshape(N_LISTS, BATCH, NEI, FEA)


# speedup vs baseline: 1.0455x; 1.0455x over previous
import functools

import jax
import jax.numpy as jnp
from jax import lax
from jax.experimental import pallas as pl
from jax.experimental.pallas import tpu as pltpu
from jax.experimental.pallas import tpu_sc as plsc

M_NUM = 6000
D_NUM = 4000
VOCAB = M_NUM + D_NUM
FEA = 128
N_LISTS = 2
BATCH = 4096
NEI = 32

_BM = 400
_NBM_M = M_NUM // _BM
_NBM_D = D_NUM // _BM

_NC = 2
_NS = 16
_NW = _NC * _NS
_B_TOT = N_LISTS * BATCH * NEI
_BPW = _B_TOT // _NW
_CH = 128


def _proj_body(m_ref, d_ref, wm_ref, wd_ref, b_ref, o_ref):
    i = pl.program_id(0)

    @pl.when(i < _NBM_M)
    def _():
        x = m_ref[...].astype(jnp.bfloat16)
        o_ref[...] = jnp.maximum(
            jnp.dot(x, wm_ref[...], preferred_element_type=jnp.float32)
            + b_ref[...],
            0.0,
        )

    @pl.when(i >= _NBM_M)
    def _():
        x = d_ref[...].astype(jnp.bfloat16)
        o_ref[...] = jnp.maximum(
            jnp.dot(x, wd_ref[...], preferred_element_type=jnp.float32)
            + b_ref[...],
            0.0,
        )


def _project_all(m_sim, d_sim, W, b):
    wm = W[:M_NUM].astype(jnp.bfloat16)
    wd = W[M_NUM:].astype(jnp.bfloat16)
    return pl.pallas_call(
        _proj_body,
        grid=(_NBM_M + _NBM_D,),
        in_specs=[
            pl.BlockSpec((_BM, M_NUM), lambda i: (jnp.minimum(i, _NBM_M - 1), 0)),
            pl.BlockSpec((_BM, D_NUM), lambda i: (jnp.clip(i - _NBM_M, 0, _NBM_D - 1), 0)),
            pl.BlockSpec((M_NUM, FEA), lambda i: (0, 0)),
            pl.BlockSpec((D_NUM, FEA), lambda i: (0, 0)),
            pl.BlockSpec((1, FEA), lambda i: (0, 0)),
        ],
        out_specs=pl.BlockSpec((_BM, FEA), lambda i: (i, 0)),
        out_shape=jax.ShapeDtypeStruct((VOCAB, FEA), jnp.float32),
    )(m_sim, d_sim, wm, wd, b.reshape(1, FEA))


_CPW = 32
_ROWS3 = _B_TOT // _CH


def _gather_body(table_hbm, idx_hbm, out_hbm, idx_v, rows_v, gsems, ssems):
    wid = lax.axis_index("s") * _NC + lax.axis_index("c")
    base3 = wid * (_BPW // _CH)
    pltpu.sync_copy(idx_hbm.at[wid], idx_v)

    def _gather_start(j, p):
        pltpu.async_copy(table_hbm.at[idx_v.at[j, 0]], rows_v.at[p, 0], gsems.at[p])
        pltpu.async_copy(table_hbm.at[idx_v.at[j, 1]], rows_v.at[p, 1], gsems.at[p])

    def _gather_wait(j, p):
        pltpu.make_async_copy(
            table_hbm.at[idx_v.at[j, 0]], rows_v.at[p, 0], gsems.at[p]
        ).wait()
        pltpu.make_async_copy(
            table_hbm.at[idx_v.at[j, 1]], rows_v.at[p, 1], gsems.at[p]
        ).wait()

    def _store_start(j, p):
        pltpu.async_copy(
            rows_v.at[p], out_hbm.at[pl.ds(base3 + 2 * j, 2)], ssems.at[p]
        )

    def _store_wait(j, p):
        pltpu.make_async_copy(
            rows_v.at[p], out_hbm.at[pl.ds(base3 + 2 * j, 2)], ssems.at[p]
        ).wait()

    _gather_start(0, 0)

    def body(t, carry):
        for p in range(3):
            j = 3 * t + p
            pn = (p + 1) % 3

            @pl.when(j >= 2)
            def _():
                _store_wait(j - 2, pn)

            _gather_start(j + 1, pn)
            _gather_wait(j, p)
            _store_start(j, p)
        return carry

    lax.fori_loop(0, (_CPW - 2) // 3, body, 0)
    _store_wait(28, 1)
    _gather_start(31, 1)
    _gather_wait(30, 0)
    _store_start(30, 0)
    _store_wait(29, 2)
    _gather_wait(31, 1)
    _store_start(31, 1)
    _store_wait(30, 0)
    _store_wait(31, 1)


_gather = functools.partial(
    pl.kernel,
    out_type=jax.ShapeDtypeStruct((_ROWS3, _CH, FEA), jnp.float32),
    mesh=plsc.VectorSubcoreMesh(core_axis_name="c", subcore_axis_name="s"),
    scratch_types=[
        pltpu.VMEM((_CPW, 2, _CH), jnp.int32),
        pltpu.VMEM((3, 2, _CH, FEA), jnp.float32),
        pltpu.SemaphoreType.DMA((3,)),
        pltpu.SemaphoreType.DMA((3,)),
    ],
)(_gather_body)


def kernel(m_sim, d_sim, nei_node_list, W, b):
    table = _project_all(m_sim, d_sim, W, b)
    idx = nei_node_list.reshape(_NW, _CPW, 2, _CH)
    out = _gather(table, idx)
    return out.reshape(N_LISTS, BATCH, NEI, FEA)
